# Initial kernel scaffold; baseline (speedup 1.0000x reference)
import jax
import jax.numpy as jnp
from jax.experimental import pallas as pl


def kernel(x, W1, b1, g1, be1, W2, b2, g2, be2, Wg, att_src, att_dst, bg, Wu1, bu1, Wu2, bu2, Wo, bo):
    raise NotImplementedError("write your pallas kernel here")


# speedup vs baseline: 12.5669x; 12.5669x over previous
import functools

import jax
import jax.numpy as jnp
from jax import lax
from jax.experimental import pallas as pl
from jax.experimental.pallas import tpu as pltpu
from jax.experimental.pallas import tpu_sc as plsc

_HID = 16
_HEADS = 4
_HD = 4
_K = 8
_N = 4096
_R = 256
_NSC = 32
_CHUNK = 512


def _conv_body(x_ref, w1_ref, b1_ref, w2_ref, b2_ref, out_ref, xpad, h1pad):
    xpad[...] = jnp.zeros((66, 66), jnp.float32)
    xpad[1:65, 1:65] = x_ref[0, 0]
    acc1 = jnp.broadcast_to(b1_ref[0][None, None, :], (64, 64, 64))
    for off in range(9):
        dy, dx = off // 3, off % 3
        acc1 = acc1 + xpad[dy:dy + 64, dx:dx + 64][..., None] * w1_ref[off][None, None, :]
    h1pad[...] = jnp.zeros((66, 66, 64), jnp.float32)
    h1pad[1:65, 1:65, :] = jnp.maximum(acc1, 0.0)
    acc2 = jnp.broadcast_to(b2_ref[0][None, :], (_N, _HID))
    for off in range(9):
        dy, dx = off // 3, off % 3
        s = h1pad[dy:dy + 64, dx:dx + 64, :].reshape(_N, 64)
        acc2 = acc2 + lax.dot_general(s, w2_ref[off],
                                      (((1,), (0,)), ((), ())),
                                      preferred_element_type=jnp.float32)
    out_ref[0] = jnp.maximum(acc2, 0.0)


def _convs(x, w1r, b1r, w2r, b2r):
    return pl.pallas_call(
        _conv_body,
        grid=(4,),
        in_specs=[
            pl.BlockSpec((1, 1, 64, 64), lambda b: (b, 0, 0, 0)),
            pl.BlockSpec((9, 64), lambda b: (0, 0)),
            pl.BlockSpec((1, 64), lambda b: (0, 0)),
            pl.BlockSpec((9, 64, 16), lambda b: (0, 0, 0)),
            pl.BlockSpec((1, 16), lambda b: (0, 0)),
        ],
        out_specs=pl.BlockSpec((1, _N, _HID), lambda b: (b, 0, 0)),
        out_shape=jax.ShapeDtypeStruct((4, _N, _HID), jnp.float32),
        scratch_shapes=[
            pltpu.VMEM((66, 66), jnp.float32),
            pltpu.VMEM((66, 66, 64), jnp.float32),
        ],
    )(x, w1r, b1r, w2r, b2r)


def _knn_body(full_ref, rows_ref, wg_ref, asrc_ref, adst_ref, nbr_ref, feat_ref):
    r = pl.program_id(1)
    nodes = full_ref[0]
    rows = rows_ref[0]
    sqf = jnp.sum(nodes * nodes, axis=1)
    sqr = jnp.sum(rows * rows, axis=1)
    g = lax.dot_general(rows, nodes, (((1,), (1,)), ((), ())),
                        preferred_element_type=jnp.float32)
    d2 = sqr[:, None] + sqf[None, :] - 2.0 * g
    col = lax.broadcasted_iota(jnp.int32, (_R, _N), 1)
    grow = r * _R + lax.broadcasted_iota(jnp.int32, (_R, _N), 0)
    d2 = jnp.where(col == grow, d2 + 1e10, d2)
    for k in range(_K):
        minv = jnp.min(d2, axis=1, keepdims=True)
        idx = jnp.min(jnp.where(d2 <= minv, col, jnp.int32(1 << 30)), axis=1)
        nbr_ref[0, k, :] = idx
        d2 = jnp.where(col == idx[:, None], jnp.float32(3e38), d2)
    xp = lax.dot_general(rows, wg_ref[...], (((1,), (0,)), ((), ())),
                         preferred_element_type=jnp.float32)
    a_s = jnp.sum(xp.reshape(_R, _HEADS, _HD) * asrc_ref[...][None], axis=2)
    a_d = jnp.sum(xp.reshape(_R, _HEADS, _HD) * adst_ref[...][None], axis=2)
    feat_ref[0] = jnp.concatenate([xp, a_s, a_d], axis=1)


def _knn(h, wg, att_src, att_dst):
    return pl.pallas_call(
        _knn_body,
        grid=(4, _N // _R),
        in_specs=[
            pl.BlockSpec((1, _N, _HID), lambda i, r: (i, 0, 0)),
            pl.BlockSpec((1, _R, _HID), lambda i, r: (i, r, 0)),
            pl.BlockSpec((16, 16), lambda i, r: (0, 0)),
            pl.BlockSpec((4, 4), lambda i, r: (0, 0)),
            pl.BlockSpec((4, 4), lambda i, r: (0, 0)),
        ],
        out_specs=[
            pl.BlockSpec((1, _K, _R), lambda i, r: (i, 0, r)),
            pl.BlockSpec((1, _R, 24), lambda i, r: (i, r, 0)),
        ],
        out_shape=[
            jax.ShapeDtypeStruct((4, _K, _N), jnp.int32),
            jax.ShapeDtypeStruct((4, _N, 24), jnp.float32),
        ],
    )(h, h, wg, att_src, att_dst)


def _gat_sc_body(feat_hbm, nbr_hbm, out_hbm, feat_v, nbr_v, out_v):
    wid = lax.axis_index("s") * 2 + lax.axis_index("c")
    img = wid // 8
    chunk = wid % 8
    pltpu.sync_copy(feat_hbm.at[img], feat_v)
    for k in range(_K):
        pltpu.sync_copy(nbr_hbm.at[img, k, pl.ds(chunk * _CHUNK, _CHUNK)],
                        nbr_v.at[pl.ds(k * _CHUNK, _CHUNK)])

    def body(gidx, _):
        li = gidx * 16 + lax.iota(jnp.int32, 16)
        gi = (chunk * _CHUNK + li) * 24
        a_d = [plsc.load_gather(feat_v, [gi + (20 + h)]) for h in range(_HEADS)]
        nks = []
        m = [jnp.full((16,), -3.4e38, jnp.float32) for _ in range(_HEADS)]
        for k in range(_K):
            nk = nbr_v[pl.ds(k * _CHUNK + gidx * 16, 16)] * 24
            nks.append(nk)
            for h in range(_HEADS):
                sv = plsc.load_gather(feat_v, [nk + (16 + h)])
                e = sv + a_d[h]
                e = jnp.where(e >= 0.0, e, 0.2 * e)
                m[h] = jnp.maximum(m[h], e)
        den = [jnp.zeros((16,), jnp.float32) for _ in range(_HEADS)]
        acc = [jnp.zeros((16,), jnp.float32) for _ in range(_HID)]
        for k in range(_K):
            nk = nks[k]
            for h in range(_HEADS):
                sv = plsc.load_gather(feat_v, [nk + (16 + h)])
                e = sv + a_d[h]
                e = jnp.where(e >= 0.0, e, 0.2 * e)
                p = jnp.exp(e - m[h])
                den[h] = den[h] + p
                for d in range(_HD):
                    f = h * _HD + d
                    msg = plsc.load_gather(feat_v, [nk + f])
                    acc[f] = acc[f] + p * msg
        for f in range(_HID):
            val = acc[f] / den[f // _HD]
            plsc.store_scatter(out_v, [li * _HID + f], val)
        return 0

    lax.fori_loop(0, _CHUNK // 16, body, 0)
    pltpu.sync_copy(out_v, out_hbm.at[img, pl.ds(chunk * _CHUNK * _HID,
                                                 _CHUNK * _HID)])


def _gat_sc(feat, nbr):
    mesh = plsc.VectorSubcoreMesh(core_axis_name="c", subcore_axis_name="s",
                                  num_cores=2, num_subcores=16)
    fn = functools.partial(
        pl.kernel,
        out_type=jax.ShapeDtypeStruct((4, _N * _HID), jnp.float32),
        mesh=mesh,
        compiler_params=pltpu.CompilerParams(needs_layout_passes=False),
        scratch_types=[
            pltpu.VMEM((_N * 24,), jnp.float32),
            pltpu.VMEM((_K * _CHUNK,), jnp.int32),
            pltpu.VMEM((_CHUNK * _HID,), jnp.float32),
        ],
    )(_gat_sc_body)
    return fn(feat.reshape(4, _N * 24), nbr).reshape(4, _N, _HID)


def _head_body(h_ref, gat_ref, mask_ref, bg_ref, wu1_ref, bu1_ref, wu2_ref,
               bu2_ref, wo_ref, bo_ref, out_ref):
    hh = h_ref[0]
    p = jnp.maximum(gat_ref[0] + bg_ref[...], 0.0)
    t = lax.dot_general(p, wu1_ref[...], (((1,), (0,)), ((), ())),
                        preferred_element_type=jnp.float32)
    t = jnp.maximum(t + bu1_ref[...], 0.0)
    u = lax.dot_general(t, wu2_ref[...], (((1,), (0,)), ((), ())),
                        preferred_element_type=jnp.float32) + bu2_ref[...]
    hm = hh + mask_ref[0] * u
    o = jnp.sum(hm * wo_ref[...], axis=1) + bo_ref[0, 0]
    out_ref[0, 0] = 1.0 / (1.0 + jnp.exp(-o))


def _head(h, gat, mask, bg, wu1r, bu1, wu2r, bu2, wo, bo):
    return pl.pallas_call(
        _head_body,
        grid=(4,),
        in_specs=[
            pl.BlockSpec((1, _N, _HID), lambda b: (b, 0, 0)),
            pl.BlockSpec((1, _N, _HID), lambda b: (b, 0, 0)),
            pl.BlockSpec((1, _N, 1), lambda b: (b, 0, 0)),
            pl.BlockSpec((1, 16), lambda b: (0, 0)),
            pl.BlockSpec((16, 128), lambda b: (0, 0)),
            pl.BlockSpec((1, 128), lambda b: (0, 0)),
            pl.BlockSpec((128, 16), lambda b: (0, 0)),
            pl.BlockSpec((1, 16), lambda b: (0, 0)),
            pl.BlockSpec((1, 16), lambda b: (0, 0)),
            pl.BlockSpec((1, 1), lambda b: (0, 0)),
        ],
        out_specs=pl.BlockSpec((1, 1, _N), lambda b: (b, 0, 0)),
        out_shape=jax.ShapeDtypeStruct((4, 1, _N), jnp.float32),
    )(h, gat, mask, bg, wu1r, bu1, wu2r, bu2, wo, bo)


def kernel(x, W1, b1, g1, be1, W2, b2, g2, be2, Wg, att_src, att_dst, bg,
           Wu1, bu1, Wu2, bu2, Wo, bo):
    s1 = g1 / jnp.sqrt(1.0 + 1e-5)
    s2 = g2 / jnp.sqrt(1.0 + 1e-5)
    w1r = jnp.transpose(W1[:, 0] * s1[:, None, None], (1, 2, 0)).reshape(9, 64)
    b1r = (b1 * s1 + be1).reshape(1, 64)
    w2r = jnp.transpose(W2 * s2[:, None, None, None], (2, 3, 1, 0)).reshape(9, 64, 16)
    b2r = (b2 * s2 + be2).reshape(1, 16)

    h = _convs(x, w1r, b1r, w2r, b2r)
    nbr, feat = _knn(h, Wg, att_src, att_dst)
    gat = _gat_sc(feat, nbr)

    mask = (jax.random.uniform(jax.random.key(42), (4, 1, 64, 64)) < 0.5)
    mask = mask.astype(jnp.float32).reshape(4, _N, 1)
    out = _head(h, gat, mask, bg.reshape(1, 16),
                jnp.transpose(Wu1[:, :, 0, 0]), bu1.reshape(1, 128),
                jnp.transpose(Wu2[:, :, 0, 0]), bu2.reshape(1, 16),
                Wo[0, :, 0, 0].reshape(1, 16), bo.reshape(1, 1))
    return out.reshape(4, 1, 64, 64)


# speedup vs baseline: 16.6700x; 1.3265x over previous
import functools

import jax
import jax.numpy as jnp
from jax import lax
from jax.experimental import pallas as pl
from jax.experimental.pallas import tpu as pltpu
from jax.experimental.pallas import tpu_sc as plsc

_HID = 16
_HEADS = 4
_HD = 4
_K = 8
_N = 4096
_R = 256
_NSC = 32
_CHUNK = 512


def _conv_body(x_ref, w1_ref, b1_ref, w2_ref, b2_ref, out_ref, xpad, h1pad):
    xpad[...] = jnp.zeros((66, 66), jnp.float32)
    xpad[1:65, 1:65] = x_ref[0, 0]
    acc1 = jnp.broadcast_to(b1_ref[0][None, None, :], (64, 64, 64))
    for off in range(9):
        dy, dx = off // 3, off % 3
        acc1 = acc1 + xpad[dy:dy + 64, dx:dx + 64][..., None] * w1_ref[off][None, None, :]
    h1pad[...] = jnp.zeros((66, 66, 64), jnp.float32)
    h1pad[1:65, 1:65, :] = jnp.maximum(acc1, 0.0)
    acc2 = jnp.broadcast_to(b2_ref[0][None, :], (_N, _HID))
    for off in range(9):
        dy, dx = off // 3, off % 3
        s = h1pad[dy:dy + 64, dx:dx + 64, :].reshape(_N, 64)
        acc2 = acc2 + lax.dot_general(s, w2_ref[off],
                                      (((1,), (0,)), ((), ())),
                                      preferred_element_type=jnp.float32)
    out_ref[0] = jnp.maximum(acc2, 0.0)


def _convs(x, w1r, b1r, w2r, b2r):
    return pl.pallas_call(
        _conv_body,
        grid=(4,),
        in_specs=[
            pl.BlockSpec((1, 1, 64, 64), lambda b: (b, 0, 0, 0)),
            pl.BlockSpec((9, 64), lambda b: (0, 0)),
            pl.BlockSpec((1, 64), lambda b: (0, 0)),
            pl.BlockSpec((9, 64, 16), lambda b: (0, 0, 0)),
            pl.BlockSpec((1, 16), lambda b: (0, 0)),
        ],
        out_specs=pl.BlockSpec((1, _N, _HID), lambda b: (b, 0, 0)),
        out_shape=jax.ShapeDtypeStruct((4, _N, _HID), jnp.float32),
        scratch_shapes=[
            pltpu.VMEM((66, 66), jnp.float32),
            pltpu.VMEM((66, 66, 64), jnp.float32),
        ],
    )(x, w1r, b1r, w2r, b2r)


def _knn_body(full_ref, rows_ref, wg_ref, asrc_ref, adst_ref, nbr_ref, feat_ref):
    r = pl.program_id(1)
    nodes = full_ref[0]
    rows = rows_ref[0]
    sqf = jnp.sum(nodes * nodes, axis=1)
    sqr = jnp.sum(rows * rows, axis=1)
    g = lax.dot_general(rows, nodes, (((1,), (1,)), ((), ())),
                        preferred_element_type=jnp.float32)
    d2 = sqr[:, None] + sqf[None, :] - 2.0 * g
    col = lax.broadcasted_iota(jnp.int32, (_R, _N), 1)
    grow = r * _R + lax.broadcasted_iota(jnp.int32, (_R, _N), 0)
    key = lax.bitcast_convert_type(d2 + 1.0, jnp.int32)
    key = (key & jnp.int32(~0xFFF)) | col
    key = jnp.where(col == grow, jnp.int32(0x7FFFFFFF), key)
    for k in range(_K):
        minv = jnp.min(key, axis=1)
        nbr_ref[0, k, :] = minv & 0xFFF
        key = jnp.where(key == minv[:, None], jnp.int32(0x7FFFFFFF), key)
    xp = lax.dot_general(rows, wg_ref[...], (((1,), (0,)), ((), ())),
                         preferred_element_type=jnp.float32)
    a_s = jnp.sum(xp.reshape(_R, _HEADS, _HD) * asrc_ref[...][None], axis=2)
    a_d = jnp.sum(xp.reshape(_R, _HEADS, _HD) * adst_ref[...][None], axis=2)
    feat_ref[0] = jnp.concatenate([xp, a_s, a_d], axis=1)


def _knn(h, wg, att_src, att_dst):
    return pl.pallas_call(
        _knn_body,
        grid=(4, _N // _R),
        in_specs=[
            pl.BlockSpec((1, _N, _HID), lambda i, r: (i, 0, 0)),
            pl.BlockSpec((1, _R, _HID), lambda i, r: (i, r, 0)),
            pl.BlockSpec((16, 16), lambda i, r: (0, 0)),
            pl.BlockSpec((4, 4), lambda i, r: (0, 0)),
            pl.BlockSpec((4, 4), lambda i, r: (0, 0)),
        ],
        out_specs=[
            pl.BlockSpec((1, _K, _R), lambda i, r: (i, 0, r)),
            pl.BlockSpec((1, _R, 24), lambda i, r: (i, r, 0)),
        ],
        out_shape=[
            jax.ShapeDtypeStruct((4, _K, _N), jnp.int32),
            jax.ShapeDtypeStruct((4, _N, 24), jnp.float32),
        ],
    )(h, h, wg, att_src, att_dst)


def _gat_sc_body(feat_hbm, nbr_hbm, out_hbm, feat_v, nbr_v, out_v):
    wid = lax.axis_index("s") * 2 + lax.axis_index("c")
    img = wid // 8
    chunk = wid % 8
    pltpu.sync_copy(feat_hbm.at[img], feat_v)
    for k in range(_K):
        pltpu.sync_copy(nbr_hbm.at[img, k, pl.ds(chunk * _CHUNK, _CHUNK)],
                        nbr_v.at[pl.ds(k * _CHUNK, _CHUNK)])

    def body(gidx, _):
        li = gidx * 16 + lax.iota(jnp.int32, 16)
        gi = (chunk * _CHUNK + li) * 24
        a_d = [plsc.load_gather(feat_v, [gi + (20 + h)]) for h in range(_HEADS)]
        nks = []
        m = [jnp.full((16,), -3.4e38, jnp.float32) for _ in range(_HEADS)]
        for k in range(_K):
            nk = nbr_v[pl.ds(k * _CHUNK + gidx * 16, 16)] * 24
            nks.append(nk)
            for h in range(_HEADS):
                sv = plsc.load_gather(feat_v, [nk + (16 + h)])
                e = sv + a_d[h]
                e = jnp.where(e >= 0.0, e, 0.2 * e)
                m[h] = jnp.maximum(m[h], e)
        den = [jnp.zeros((16,), jnp.float32) for _ in range(_HEADS)]
        acc = [jnp.zeros((16,), jnp.float32) for _ in range(_HID)]
        for k in range(_K):
            nk = nks[k]
            for h in range(_HEADS):
                sv = plsc.load_gather(feat_v, [nk + (16 + h)])
                e = sv + a_d[h]
                e = jnp.where(e >= 0.0, e, 0.2 * e)
                p = jnp.exp(e - m[h])
                den[h] = den[h] + p
                for d in range(_HD):
                    f = h * _HD + d
                    msg = plsc.load_gather(feat_v, [nk + f])
                    acc[f] = acc[f] + p * msg
        for f in range(_HID):
            val = acc[f] / den[f // _HD]
            plsc.store_scatter(out_v, [li * _HID + f], val)
        return 0

    lax.fori_loop(0, _CHUNK // 16, body, 0)
    pltpu.sync_copy(out_v, out_hbm.at[img, pl.ds(chunk * _CHUNK * _HID,
                                                 _CHUNK * _HID)])


def _gat_sc(feat, nbr):
    mesh = plsc.VectorSubcoreMesh(core_axis_name="c", subcore_axis_name="s",
                                  num_cores=2, num_subcores=16)
    fn = functools.partial(
        pl.kernel,
        out_type=jax.ShapeDtypeStruct((4, _N * _HID), jnp.float32),
        mesh=mesh,
        compiler_params=pltpu.CompilerParams(needs_layout_passes=False),
        scratch_types=[
            pltpu.VMEM((_N * 24,), jnp.float32),
            pltpu.VMEM((_K * _CHUNK,), jnp.int32),
            pltpu.VMEM((_CHUNK * _HID,), jnp.float32),
        ],
    )(_gat_sc_body)
    return fn(feat.reshape(4, _N * 24), nbr).reshape(4, _N, _HID)


def _head_body(h_ref, gat_ref, mask_ref, bg_ref, wu1_ref, bu1_ref, wu2_ref,
               bu2_ref, wo_ref, bo_ref, out_ref):
    hh = h_ref[0]
    p = jnp.maximum(gat_ref[0] + bg_ref[...], 0.0)
    t = lax.dot_general(p, wu1_ref[...], (((1,), (0,)), ((), ())),
                        preferred_element_type=jnp.float32)
    t = jnp.maximum(t + bu1_ref[...], 0.0)
    u = lax.dot_general(t, wu2_ref[...], (((1,), (0,)), ((), ())),
                        preferred_element_type=jnp.float32) + bu2_ref[...]
    hm = hh + mask_ref[0] * u
    o = jnp.sum(hm * wo_ref[...], axis=1) + bo_ref[0, 0]
    out_ref[0, 0] = 1.0 / (1.0 + jnp.exp(-o))


def _head(h, gat, mask, bg, wu1r, bu1, wu2r, bu2, wo, bo):
    return pl.pallas_call(
        _head_body,
        grid=(4,),
        in_specs=[
            pl.BlockSpec((1, _N, _HID), lambda b: (b, 0, 0)),
            pl.BlockSpec((1, _N, _HID), lambda b: (b, 0, 0)),
            pl.BlockSpec((1, _N, 1), lambda b: (b, 0, 0)),
            pl.BlockSpec((1, 16), lambda b: (0, 0)),
            pl.BlockSpec((16, 128), lambda b: (0, 0)),
            pl.BlockSpec((1, 128), lambda b: (0, 0)),
            pl.BlockSpec((128, 16), lambda b: (0, 0)),
            pl.BlockSpec((1, 16), lambda b: (0, 0)),
            pl.BlockSpec((1, 16), lambda b: (0, 0)),
            pl.BlockSpec((1, 1), lambda b: (0, 0)),
        ],
        out_specs=pl.BlockSpec((1, 1, _N), lambda b: (b, 0, 0)),
        out_shape=jax.ShapeDtypeStruct((4, 1, _N), jnp.float32),
    )(h, gat, mask, bg, wu1r, bu1, wu2r, bu2, wo, bo)


def kernel(x, W1, b1, g1, be1, W2, b2, g2, be2, Wg, att_src, att_dst, bg,
           Wu1, bu1, Wu2, bu2, Wo, bo):
    s1 = g1 / jnp.sqrt(1.0 + 1e-5)
    s2 = g2 / jnp.sqrt(1.0 + 1e-5)
    w1r = jnp.transpose(W1[:, 0] * s1[:, None, None], (1, 2, 0)).reshape(9, 64)
    b1r = (b1 * s1 + be1).reshape(1, 64)
    w2r = jnp.transpose(W2 * s2[:, None, None, None], (2, 3, 1, 0)).reshape(9, 64, 16)
    b2r = (b2 * s2 + be2).reshape(1, 16)

    h = _convs(x, w1r, b1r, w2r, b2r)
    nbr, feat = _knn(h, Wg, att_src, att_dst)
    gat = _gat_sc(feat, nbr)

    mask = (jax.random.uniform(jax.random.key(42), (4, 1, 64, 64)) < 0.5)
    mask = mask.astype(jnp.float32).reshape(4, _N, 1)
    out = _head(h, gat, mask, bg.reshape(1, 16),
                jnp.transpose(Wu1[:, :, 0, 0]), bu1.reshape(1, 128),
                jnp.transpose(Wu2[:, :, 0, 0]), bu2.reshape(1, 16),
                Wo[0, :, 0, 0].reshape(1, 16), bo.reshape(1, 1))
    return out.reshape(4, 1, 64, 64)


# speedup vs baseline: 19.4608x; 1.1674x over previous
import functools

import jax
import jax.numpy as jnp
from jax import lax
from jax.experimental import pallas as pl
from jax.experimental.pallas import tpu as pltpu
from jax.experimental.pallas import tpu_sc as plsc

_HID = 16
_HEADS = 4
_HD = 4
_K = 8
_N = 4096
_R = 256
_NSC = 32
_CHUNK = 512


def _conv_body(x_ref, w1_ref, b1_ref, w2_ref, b2_ref, out_ref, xpad, h1pad):
    xpad[...] = jnp.zeros((66, 66), jnp.float32)
    xpad[1:65, 1:65] = x_ref[0, 0]
    acc1 = jnp.broadcast_to(b1_ref[0][None, None, :], (64, 64, 64))
    for off in range(9):
        dy, dx = off // 3, off % 3
        acc1 = acc1 + xpad[dy:dy + 64, dx:dx + 64][..., None] * w1_ref[off][None, None, :]
    h1pad[...] = jnp.zeros((66, 66, 64), jnp.float32)
    h1pad[1:65, 1:65, :] = jnp.maximum(acc1, 0.0)
    acc2 = jnp.broadcast_to(b2_ref[0][None, :], (_N, _HID))
    for off in range(9):
        dy, dx = off // 3, off % 3
        s = h1pad[dy:dy + 64, dx:dx + 64, :].reshape(_N, 64)
        acc2 = acc2 + lax.dot_general(s, w2_ref[off],
                                      (((1,), (0,)), ((), ())),
                                      preferred_element_type=jnp.float32)
    out_ref[0] = jnp.maximum(acc2, 0.0)


def _convs(x, w1r, b1r, w2r, b2r):
    return pl.pallas_call(
        _conv_body,
        grid=(4,),
        in_specs=[
            pl.BlockSpec((1, 1, 64, 64), lambda b: (b, 0, 0, 0)),
            pl.BlockSpec((9, 64), lambda b: (0, 0)),
            pl.BlockSpec((1, 64), lambda b: (0, 0)),
            pl.BlockSpec((9, 64, 16), lambda b: (0, 0, 0)),
            pl.BlockSpec((1, 16), lambda b: (0, 0)),
        ],
        out_specs=pl.BlockSpec((1, _N, _HID), lambda b: (b, 0, 0)),
        out_shape=jax.ShapeDtypeStruct((4, _N, _HID), jnp.float32),
        scratch_shapes=[
            pltpu.VMEM((66, 66), jnp.float32),
            pltpu.VMEM((66, 66, 64), jnp.float32),
        ],
    )(x, w1r, b1r, w2r, b2r)


def _knn_body(full_ref, rows_ref, wg_ref, asrc_ref, adst_ref, nbr_ref, feat_ref):
    r = pl.program_id(1)
    nodes = full_ref[0]
    rows = rows_ref[0]
    sqf = jnp.sum(nodes * nodes, axis=1)
    sqr = jnp.sum(rows * rows, axis=1)
    g = lax.dot_general(rows, nodes, (((1,), (1,)), ((), ())),
                        preferred_element_type=jnp.float32)
    d2 = sqr[:, None] + sqf[None, :] - 2.0 * g
    col = lax.broadcasted_iota(jnp.int32, (_R, _N), 1)
    grow = r * _R + lax.broadcasted_iota(jnp.int32, (_R, _N), 0)
    big = jnp.float32(3.4028235e38)
    key = lax.bitcast_convert_type(d2 + 1.0, jnp.int32)
    key = (key & jnp.int32(~0xFFF)) | col
    keyf = lax.bitcast_convert_type(key, jnp.float32)
    keyf = jnp.where(col == grow, big, keyf)
    m = jnp.min(keyf, axis=1, keepdims=True)
    nbr_ref[0, 0, :] = lax.bitcast_convert_type(m[:, 0], jnp.int32) & 0xFFF
    for k in range(1, _K):
        m = jnp.min(jnp.where(keyf > m, keyf, big), axis=1, keepdims=True)
        nbr_ref[0, k, :] = lax.bitcast_convert_type(m[:, 0], jnp.int32) & 0xFFF
    xp = lax.dot_general(rows, wg_ref[...], (((1,), (0,)), ((), ())),
                         preferred_element_type=jnp.float32)
    a_s = jnp.sum(xp.reshape(_R, _HEADS, _HD) * asrc_ref[...][None], axis=2)
    a_d = jnp.sum(xp.reshape(_R, _HEADS, _HD) * adst_ref[...][None], axis=2)
    feat_ref[0] = jnp.concatenate([xp, a_s, a_d], axis=1)


def _knn(h, wg, att_src, att_dst):
    return pl.pallas_call(
        _knn_body,
        grid=(4, _N // _R),
        in_specs=[
            pl.BlockSpec((1, _N, _HID), lambda i, r: (i, 0, 0)),
            pl.BlockSpec((1, _R, _HID), lambda i, r: (i, r, 0)),
            pl.BlockSpec((16, 16), lambda i, r: (0, 0)),
            pl.BlockSpec((4, 4), lambda i, r: (0, 0)),
            pl.BlockSpec((4, 4), lambda i, r: (0, 0)),
        ],
        out_specs=[
            pl.BlockSpec((1, _K, _R), lambda i, r: (i, 0, r)),
            pl.BlockSpec((1, _R, 24), lambda i, r: (i, r, 0)),
        ],
        out_shape=[
            jax.ShapeDtypeStruct((4, _K, _N), jnp.int32),
            jax.ShapeDtypeStruct((4, _N, 24), jnp.float32),
        ],
    )(h, h, wg, att_src, att_dst)


def _gat_sc_body(feat_hbm, nbr_hbm, out_hbm, feat_v, nbr_v, out_v):
    wid = lax.axis_index("s") * 2 + lax.axis_index("c")
    img = wid // 8
    chunk = wid % 8
    pltpu.sync_copy(feat_hbm.at[img], feat_v)
    for k in range(_K):
        pltpu.sync_copy(nbr_hbm.at[img, k, pl.ds(chunk * _CHUNK, _CHUNK)],
                        nbr_v.at[pl.ds(k * _CHUNK, _CHUNK)])

    def body(gidx, _):
        li = gidx * 16 + lax.iota(jnp.int32, 16)
        gi = (chunk * _CHUNK + li) * 24
        a_d = [plsc.load_gather(feat_v, [gi + (20 + h)]) for h in range(_HEADS)]
        nks = []
        m = [jnp.full((16,), -3.4e38, jnp.float32) for _ in range(_HEADS)]
        for k in range(_K):
            nk = nbr_v[pl.ds(k * _CHUNK + gidx * 16, 16)] * 24
            nks.append(nk)
            for h in range(_HEADS):
                sv = plsc.load_gather(feat_v, [nk + (16 + h)])
                e = sv + a_d[h]
                e = jnp.where(e >= 0.0, e, 0.2 * e)
                m[h] = jnp.maximum(m[h], e)
        den = [jnp.zeros((16,), jnp.float32) for _ in range(_HEADS)]
        acc = [jnp.zeros((16,), jnp.float32) for _ in range(_HID)]
        for k in range(_K):
            nk = nks[k]
            for h in range(_HEADS):
                sv = plsc.load_gather(feat_v, [nk + (16 + h)])
                e = sv + a_d[h]
                e = jnp.where(e >= 0.0, e, 0.2 * e)
                p = jnp.exp(e - m[h])
                den[h] = den[h] + p
                for d in range(_HD):
                    f = h * _HD + d
                    msg = plsc.load_gather(feat_v, [nk + f])
                    acc[f] = acc[f] + p * msg
        for f in range(_HID):
            val = acc[f] / den[f // _HD]
            plsc.store_scatter(out_v, [li * _HID + f], val)
        return 0

    lax.fori_loop(0, _CHUNK // 16, body, 0)
    pltpu.sync_copy(out_v, out_hbm.at[img, pl.ds(chunk * _CHUNK * _HID,
                                                 _CHUNK * _HID)])


def _gat_sc(feat, nbr):
    mesh = plsc.VectorSubcoreMesh(core_axis_name="c", subcore_axis_name="s",
                                  num_cores=2, num_subcores=16)
    fn = functools.partial(
        pl.kernel,
        out_type=jax.ShapeDtypeStruct((4, _N * _HID), jnp.float32),
        mesh=mesh,
        compiler_params=pltpu.CompilerParams(needs_layout_passes=False),
        scratch_types=[
            pltpu.VMEM((_N * 24,), jnp.float32),
            pltpu.VMEM((_K * _CHUNK,), jnp.int32),
            pltpu.VMEM((_CHUNK * _HID,), jnp.float32),
        ],
    )(_gat_sc_body)
    return fn(feat.reshape(4, _N * 24), nbr).reshape(4, _N, _HID)


def _head_body(h_ref, gat_ref, mask_ref, bg_ref, wu1_ref, bu1_ref, wu2_ref,
               bu2_ref, wo_ref, bo_ref, out_ref):
    hh = h_ref[0]
    p = jnp.maximum(gat_ref[0] + bg_ref[...], 0.0)
    t = lax.dot_general(p, wu1_ref[...], (((1,), (0,)), ((), ())),
                        preferred_element_type=jnp.float32)
    t = jnp.maximum(t + bu1_ref[...], 0.0)
    u = lax.dot_general(t, wu2_ref[...], (((1,), (0,)), ((), ())),
                        preferred_element_type=jnp.float32) + bu2_ref[...]
    hm = hh + mask_ref[0] * u
    o = jnp.sum(hm * wo_ref[...], axis=1) + bo_ref[0, 0]
    out_ref[0, 0] = 1.0 / (1.0 + jnp.exp(-o))


def _head(h, gat, mask, bg, wu1r, bu1, wu2r, bu2, wo, bo):
    return pl.pallas_call(
        _head_body,
        grid=(4,),
        in_specs=[
            pl.BlockSpec((1, _N, _HID), lambda b: (b, 0, 0)),
            pl.BlockSpec((1, _N, _HID), lambda b: (b, 0, 0)),
            pl.BlockSpec((1, _N, 1), lambda b: (b, 0, 0)),
            pl.BlockSpec((1, 16), lambda b: (0, 0)),
            pl.BlockSpec((16, 128), lambda b: (0, 0)),
            pl.BlockSpec((1, 128), lambda b: (0, 0)),
            pl.BlockSpec((128, 16), lambda b: (0, 0)),
            pl.BlockSpec((1, 16), lambda b: (0, 0)),
            pl.BlockSpec((1, 16), lambda b: (0, 0)),
            pl.BlockSpec((1, 1), lambda b: (0, 0)),
        ],
        out_specs=pl.BlockSpec((1, 1, _N), lambda b: (b, 0, 0)),
        out_shape=jax.ShapeDtypeStruct((4, 1, _N), jnp.float32),
    )(h, gat, mask, bg, wu1r, bu1, wu2r, bu2, wo, bo)


def kernel(x, W1, b1, g1, be1, W2, b2, g2, be2, Wg, att_src, att_dst, bg,
           Wu1, bu1, Wu2, bu2, Wo, bo):
    s1 = g1 / jnp.sqrt(1.0 + 1e-5)
    s2 = g2 / jnp.sqrt(1.0 + 1e-5)
    w1r = jnp.transpose(W1[:, 0] * s1[:, None, None], (1, 2, 0)).reshape(9, 64)
    b1r = (b1 * s1 + be1).reshape(1, 64)
    w2r = jnp.transpose(W2 * s2[:, None, None, None], (2, 3, 1, 0)).reshape(9, 64, 16)
    b2r = (b2 * s2 + be2).reshape(1, 16)

    h = _convs(x, w1r, b1r, w2r, b2r)
    nbr, feat = _knn(h, Wg, att_src, att_dst)
    gat = _gat_sc(feat, nbr)

    mask = (jax.random.uniform(jax.random.key(42), (4, 1, 64, 64)) < 0.5)
    mask = mask.astype(jnp.float32).reshape(4, _N, 1)
    out = _head(h, gat, mask, bg.reshape(1, 16),
                jnp.transpose(Wu1[:, :, 0, 0]), bu1.reshape(1, 128),
                jnp.transpose(Wu2[:, :, 0, 0]), bu2.reshape(1, 16),
                Wo[0, :, 0, 0].reshape(1, 16), bo.reshape(1, 1))
    return out.reshape(4, 1, 64, 64)
